# baseline (device time: 56540 ns/iter reference)
import jax
import jax.numpy as jnp
from jax import lax
from jax.experimental import pallas as pl
from jax.experimental.pallas import tpu as pltpu

N_DEV = 4
GELU_C = 0.7978845608028654


def _gelu(y):
    return 0.5 * y * (1.0 + jnp.tanh(GELU_C * (y + 0.044715 * y * y * y)))


def kernel(x, w_mat):
    m_per, k = x.shape
    _, n = w_mat.shape
    n_per = n // N_DEV

    def body(x_ref, w_ref, out_ref, y_send, recv_buf, send_sems, recv_sems):
        d = pl.program_id(0)
        my = lax.axis_index("i")

        @pl.when(d == 0)
        def _():
            barrier = pltpu.get_barrier_semaphore()
            for dd in range(1, N_DEV):
                pl.semaphore_signal(
                    barrier, inc=1,
                    device_id=((my + dd) % N_DEV,),
                    device_id_type=pl.DeviceIdType.MESH,
                )
            pl.semaphore_wait(barrier, N_DEV - 1)

        y = _gelu(
            jnp.dot(x_ref[...], w_ref[...], preferred_element_type=jnp.float32)
        )

        @pl.when(d < N_DEV - 1)
        def _():
            tgt = (my + 1 + d) % N_DEV
            y_send[d] = y.astype(jnp.bfloat16)
            rdma = pltpu.make_async_remote_copy(
                src_ref=y_send.at[d],
                dst_ref=recv_buf.at[my],
                send_sem=send_sems.at[d],
                recv_sem=recv_sems.at[my],
                device_id=(tgt,),
                device_id_type=pl.DeviceIdType.MESH,
            )
            rdma.start()

        @pl.when(d == N_DEV - 1)
        def _():
            out_ref[pl.ds(my * m_per, m_per), :] = y
            for dd in range(1, N_DEV):
                src = (my - dd) % N_DEV
                recv = pltpu.make_async_remote_copy(
                    src_ref=y_send.at[0],
                    dst_ref=recv_buf.at[src],
                    send_sem=send_sems.at[0],
                    recv_sem=recv_sems.at[src],
                    device_id=(src,),
                    device_id_type=pl.DeviceIdType.MESH,
                )
                recv.wait_recv()
                out_ref[pl.ds(src * m_per, m_per), :] = recv_buf[
                    src
                ].astype(jnp.float32)
            for dd in range(N_DEV - 1):
                snd = pltpu.make_async_remote_copy(
                    src_ref=y_send.at[dd],
                    dst_ref=recv_buf.at[0],
                    send_sem=send_sems.at[dd],
                    recv_sem=recv_sems.at[0],
                    device_id=((my + 1 + dd) % N_DEV,),
                    device_id_type=pl.DeviceIdType.MESH,
                )
                snd.wait_send()

    def w_index(d):
        return (0, (lax.axis_index("i") + 1 + d) % N_DEV)

    return pl.pallas_call(
        body,
        grid=(N_DEV,),
        in_specs=[
            pl.BlockSpec(memory_space=pltpu.VMEM),
            pl.BlockSpec((k, n_per), w_index),
        ],
        out_specs=pl.BlockSpec(memory_space=pltpu.VMEM),
        out_shape=jax.ShapeDtypeStruct((N_DEV * m_per, n_per), jnp.float32),
        scratch_shapes=[
            pltpu.VMEM((N_DEV - 1, m_per, n_per), jnp.bfloat16),
            pltpu.VMEM((N_DEV, m_per, n_per), jnp.bfloat16),
            pltpu.SemaphoreType.DMA((N_DEV - 1,)),
            pltpu.SemaphoreType.DMA((N_DEV,)),
        ],
        compiler_params=pltpu.CompilerParams(
            collective_id=0,
            dimension_semantics=("arbitrary",),
            vmem_limit_bytes=60 * 1024 * 1024,
        ),
    )(x, w_mat)


# device time: 54398 ns/iter; 1.0394x vs baseline; 1.0394x over previous
import jax
import jax.numpy as jnp
from jax import lax
from jax.experimental import pallas as pl
from jax.experimental.pallas import tpu as pltpu

N_DEV = 4
N_HALF = 2
GELU_C = 0.7978845608028654


def _gelu(y):
    return 0.5 * y * (1.0 + jnp.tanh(GELU_C * (y + 0.044715 * y * y * y)))


def kernel(x, w_mat):
    m_per, k = x.shape
    _, n = w_mat.shape
    n_per = n // N_DEV
    m_half = m_per // N_HALF

    def body(x_ref, w_ref, out_ref, y_send, recv_buf, send_sems, recv_sems):
        d = pl.program_id(0)
        my = lax.axis_index("i")

        @pl.when(d == 0)
        def _():
            barrier = pltpu.get_barrier_semaphore()
            for dd in range(1, N_DEV):
                pl.semaphore_signal(
                    barrier, inc=1,
                    device_id=((my + dd) % N_DEV,),
                    device_id_type=pl.DeviceIdType.MESH,
                )
            pl.semaphore_wait(barrier, N_DEV - 1)

        @pl.when(d < N_DEV - 1)
        def _():
            tgt = (my + 1 + d) % N_DEV
            for h in range(N_HALF):
                rows = pl.ds(h * m_half, m_half)
                y = _gelu(
                    jnp.dot(
                        x_ref[rows, :], w_ref[...],
                        preferred_element_type=jnp.float32,
                    )
                )
                y_send[d, rows, :] = y.astype(jnp.bfloat16)
                rdma = pltpu.make_async_remote_copy(
                    src_ref=y_send.at[d, rows, :],
                    dst_ref=recv_buf.at[my, rows, :],
                    send_sem=send_sems.at[d, h],
                    recv_sem=recv_sems.at[my],
                    device_id=(tgt,),
                    device_id_type=pl.DeviceIdType.MESH,
                )
                rdma.start()

        @pl.when(d == N_DEV - 1)
        def _():
            y = _gelu(
                jnp.dot(
                    x_ref[...], w_ref[...],
                    preferred_element_type=jnp.float32,
                )
            )
            out_ref[pl.ds(my * m_per, m_per), :] = y
            for dd in range(1, N_DEV):
                src = (my - dd) % N_DEV
                recv = pltpu.make_async_remote_copy(
                    src_ref=y_send.at[0],
                    dst_ref=recv_buf.at[src],
                    send_sem=send_sems.at[0, 0],
                    recv_sem=recv_sems.at[src],
                    device_id=(src,),
                    device_id_type=pl.DeviceIdType.MESH,
                )
                recv.wait_recv()
                out_ref[pl.ds(src * m_per, m_per), :] = recv_buf[
                    src
                ].astype(jnp.float32)
            for dd in range(N_DEV - 1):
                for h in range(N_HALF):
                    rows = pl.ds(h * m_half, m_half)
                    snd = pltpu.make_async_remote_copy(
                        src_ref=y_send.at[dd, rows, :],
                        dst_ref=recv_buf.at[0, rows, :],
                        send_sem=send_sems.at[dd, h],
                        recv_sem=recv_sems.at[0],
                        device_id=((my + 1 + dd) % N_DEV,),
                        device_id_type=pl.DeviceIdType.MESH,
                    )
                    snd.wait_send()

    def w_index(d):
        return (0, (lax.axis_index("i") + 1 + d) % N_DEV)

    return pl.pallas_call(
        body,
        grid=(N_DEV,),
        in_specs=[
            pl.BlockSpec(memory_space=pltpu.VMEM),
            pl.BlockSpec((k, n_per), w_index),
        ],
        out_specs=pl.BlockSpec(memory_space=pltpu.VMEM),
        out_shape=jax.ShapeDtypeStruct((N_DEV * m_per, n_per), jnp.float32),
        scratch_shapes=[
            pltpu.VMEM((N_DEV - 1, m_per, n_per), jnp.bfloat16),
            pltpu.VMEM((N_DEV, m_per, n_per), jnp.bfloat16),
            pltpu.SemaphoreType.DMA((N_DEV - 1, N_HALF)),
            pltpu.SemaphoreType.DMA((N_DEV,)),
        ],
        compiler_params=pltpu.CompilerParams(
            collective_id=0,
            dimension_semantics=("arbitrary",),
            vmem_limit_bytes=60 * 1024 * 1024,
        ),
    )(x, w_mat)
